# baseline (speedup 1.0000x reference)
import functools

import jax
import jax.numpy as jnp
from jax import lax
from jax.experimental import pallas as pl
from jax.experimental.pallas import tpu as pltpu
from jax.experimental.pallas import tpu_sc as plsc

_FEATURE_DIM = 128
_NLANES = 16


def _feature_kernel(num_hbm, table_hbm, out_hbm, num_v, row_v):
    wid = lax.axis_index("s") * 2 + lax.axis_index("c")

    @pl.when(wid == 0)
    def _():
        pltpu.sync_copy(num_hbm, num_v)
        n = num_v[...][0]
        idx = jnp.int32(0)
        for b in (1, 2, 3, 4, 8, 16, 32, 64):
            idx = idx + jnp.where(n >= b, jnp.int32(1), jnp.int32(0))
        pltpu.sync_copy(table_hbm.at[idx], row_v)
        pltpu.sync_copy(row_v, out_hbm)


def kernel(num, table):
    num_vec = jnp.full((_NLANES,), num, dtype=jnp.int32)
    mesh = plsc.VectorSubcoreMesh(core_axis_name="c", subcore_axis_name="s")
    run = functools.partial(
        pl.kernel,
        out_type=jax.ShapeDtypeStruct((_FEATURE_DIM,), jnp.float32),
        mesh=mesh,
        scratch_types=[
            pltpu.VMEM((_NLANES,), jnp.int32),
            pltpu.VMEM((_FEATURE_DIM,), jnp.float32),
        ],
    )(_feature_kernel)
    return run(num_vec, table)


# speedup vs baseline: 1.0749x; 1.0749x over previous
import functools

import jax
import jax.numpy as jnp
from jax import lax
from jax.experimental import pallas as pl
from jax.experimental.pallas import tpu as pltpu
from jax.experimental.pallas import tpu_sc as plsc

_FEATURE_DIM = 128
_NLANES = 16


def _feature_kernel(num_hbm, table_hbm, out_hbm, num_v):
    @pl.when(lax.axis_index("s") == 0)
    def _():
        pltpu.sync_copy(num_hbm, num_v.at[pl.ds(0, 1)])
        n = num_v[...][0]
        idx = jnp.int32(0)
        for b in (1, 2, 3, 4, 8, 16, 32, 64):
            idx = idx + jnp.where(n >= b, jnp.int32(1), jnp.int32(0))
        pltpu.sync_copy(table_hbm.at[idx], out_hbm)


def kernel(num, table):
    num_vec = jnp.asarray(num, dtype=jnp.int32).reshape((1,))
    mesh = plsc.VectorSubcoreMesh(
        core_axis_name="c", subcore_axis_name="s", num_cores=1)
    run = functools.partial(
        pl.kernel,
        out_type=jax.ShapeDtypeStruct((_FEATURE_DIM,), jnp.float32),
        mesh=mesh,
        scratch_types=[pltpu.VMEM((_NLANES,), jnp.int32)],
    )(_feature_kernel)
    return run(num_vec, table)


# speedup vs baseline: 1.1676x; 1.0862x over previous
import functools

import jax
import jax.numpy as jnp
from jax import lax
from jax.experimental import pallas as pl
from jax.experimental.pallas import tpu as pltpu
from jax.experimental.pallas import tpu_sc as plsc

_FEATURE_DIM = 128


def _feature_kernel(num_hbm, table_hbm, out_hbm, num_s):
    @pl.when(lax.axis_index("c") == 0)
    def _():
        pltpu.sync_copy(num_hbm, num_s)
        n = num_s[0]
        idx = jnp.int32(0)
        for b in (1, 2, 3, 4, 8, 16, 32, 64):
            idx = idx + jnp.where(n >= b, jnp.int32(1), jnp.int32(0))
        pltpu.sync_copy(table_hbm.at[idx], out_hbm)


def kernel(num, table):
    num_vec = jnp.asarray(num, dtype=jnp.int32).reshape((1,))
    mesh = plsc.ScalarSubcoreMesh(axis_name="c", num_cores=1)
    run = functools.partial(
        pl.kernel,
        out_type=jax.ShapeDtypeStruct((_FEATURE_DIM,), jnp.float32),
        mesh=mesh,
        scratch_types=[pltpu.SMEM((1,), jnp.int32)],
    )(_feature_kernel)
    return run(num_vec, table)


# speedup vs baseline: 1.1699x; 1.0020x over previous
import functools

import jax
import jax.numpy as jnp
from jax.experimental import pallas as pl
from jax.experimental.pallas import tpu as pltpu
from jax.experimental.pallas import tpu_sc as plsc

_FEATURE_DIM = 128
_NROWS = 9


def _feature_kernel(num_hbm, table_hbm, out_hbm, num_s, table_vs, sem_n, sem_t):
    cp_n = pltpu.make_async_copy(num_hbm, num_s, sem_n)
    cp_t = pltpu.make_async_copy(table_hbm, table_vs, sem_t)
    cp_n.start()
    cp_t.start()
    cp_n.wait()
    n = num_s[0]
    idx = jnp.int32(0)
    for b in (1, 2, 3, 4, 8, 16, 32, 64):
        idx = idx + jnp.where(n >= b, jnp.int32(1), jnp.int32(0))
    cp_t.wait()
    pltpu.sync_copy(table_vs.at[idx], out_hbm)


def kernel(num, table):
    num_vec = jnp.asarray(num, dtype=jnp.int32).reshape((1,))
    mesh = plsc.ScalarSubcoreMesh(axis_name="c", num_cores=1)
    run = functools.partial(
        pl.kernel,
        out_type=jax.ShapeDtypeStruct((_FEATURE_DIM,), jnp.float32),
        mesh=mesh,
        scratch_types=[
            pltpu.SMEM((1,), jnp.int32),
            pltpu.VMEM_SHARED((_NROWS, _FEATURE_DIM), jnp.float32),
            pltpu.SemaphoreType.DMA,
            pltpu.SemaphoreType.DMA,
        ],
    )(_feature_kernel)
    return run(num_vec, table)
